# Initial kernel scaffold; baseline (speedup 1.0000x reference)
import jax
import jax.numpy as jnp
from jax.experimental import pallas as pl


def kernel(users_feature, items_feature, bundles_feature, ui_src, ui_dst, bi_src, bi_dst, ub_src, ub_dst, users_idx, bundles_idx):
    raise NotImplementedError("write your pallas kernel here")


# speedup vs baseline: 11.9161x; 11.9161x over previous
import functools
import math

import jax
import jax.numpy as jnp
from jax import lax
from jax.experimental import pallas as pl
from jax.experimental.pallas import tpu as pltpu
from jax.experimental.pallas import tpu_sc as plsc

NC = 2
NS = 16
NW = NC * NS
L = 16
D = 64
K = 128

N_U = 20000
N_I = 20000
N_B = 10000
BATCH = 4096


def _pad(n):
    m = 8 * NW
    return ((n + m - 1) // m) * m


N_UP = _pad(N_U)
N_IP = _pad(N_I)
N_BP = _pad(N_B)

_MESH = plsc.VectorSubcoreMesh(core_axis_name="c", subcore_axis_name="s")
_SC_PARAMS = pltpu.CompilerParams(use_tc_tiling_on_sc=False)


def _wid():
    return lax.axis_index("s") * NC + lax.axis_index("c")


def _zero_fill(zbuf, rows, cols):
    zv = jnp.zeros((L,), jnp.float32)
    for i in range(rows):
        for j in range(cols // L):
            zbuf[i, pl.ds(j * L, L)] = zv


def _zero_region(zbuf, zrows, acc, base, total):
    full, rem = divmod(total, zrows)
    for t in range(full):
        pltpu.sync_copy(zbuf, acc.at[pl.ds(base + t * zrows, zrows)])
    if rem:
        pltpu.sync_copy(zbuf.at[pl.ds(0, rem)],
                        acc.at[pl.ds(base + full * zrows, rem)])


@functools.cache
def _deg_kernel(E, napad, nbpad):
    e_per = E // NW
    n_full, rem = divmod(e_per, K)
    za = napad // NS
    zb = nbpad // NS
    ZR = 128

    @functools.partial(
        pl.kernel,
        out_type=(jax.ShapeDtypeStruct((NC * napad, L), jnp.float32),
                  jax.ShapeDtypeStruct((NC * nbpad, L), jnp.float32)),
        mesh=_MESH,
        scratch_types=[
            pltpu.VMEM_SHARED((napad, L), jnp.float32),
            pltpu.VMEM_SHARED((nbpad, L), jnp.float32),
            pltpu.VMEM((ZR, L), jnp.float32),
            pltpu.VMEM((K, L), jnp.float32),
            pltpu.VMEM((K,), jnp.int32),
            pltpu.VMEM((K,), jnp.int32),
        ] + ([pltpu.VMEM((rem,), jnp.int32), pltpu.VMEM((rem,), jnp.int32)]
             if rem else []),
        compiler_params=_SC_PARAMS,
    )
    def body(src_hbm, dst_hbm, outa, outb, acc_a, acc_b, zbuf, ones_v,
             si, di, *rembufs):
        c = lax.axis_index("c")
        s = lax.axis_index("s")
        wid = s * NC + c
        _zero_fill(zbuf, ZR, L)
        ov = jnp.full((L,), 1.0, jnp.float32)
        for i in range(K):
            ones_v[i, :] = ov
        _zero_region(zbuf, ZR, acc_a, s * za, za)
        _zero_region(zbuf, ZR, acc_b, s * zb, zb)
        plsc.subcore_barrier()

        def chunk(j, _):
            base = wid * e_per + j * K
            pltpu.sync_copy(src_hbm.at[pl.ds(base, K)], si)
            pltpu.sync_copy(dst_hbm.at[pl.ds(base, K)], di)
            pltpu.sync_copy(ones_v, acc_a.at[si], add=True)
            pltpu.sync_copy(ones_v, acc_b.at[di], add=True)
            return 0

        lax.fori_loop(0, n_full, chunk, 0)
        if rem:
            sr, dr = rembufs
            base = wid * e_per + n_full * K
            pltpu.sync_copy(src_hbm.at[pl.ds(base, rem)], sr)
            pltpu.sync_copy(dst_hbm.at[pl.ds(base, rem)], dr)
            pltpu.sync_copy(ones_v.at[pl.ds(0, rem)], acc_a.at[sr], add=True)
            pltpu.sync_copy(ones_v.at[pl.ds(0, rem)], acc_b.at[dr], add=True)
        plsc.subcore_barrier()
        pltpu.sync_copy(acc_a.at[pl.ds(s * za, za)],
                        outa.at[pl.ds(c * napad + s * za, za)])
        pltpu.sync_copy(acc_b.at[pl.ds(s * zb, zb)],
                        outb.at[pl.ds(c * nbpad + s * zb, zb)])

    return body


@functools.cache
def _edge_kernel(E, tabrows, ndstpad):
    e_per = E // NW
    n_full, rem = divmod(e_per, K)
    zr = ndstpad // NS
    ZR = 128

    @functools.partial(
        pl.kernel,
        out_type=jax.ShapeDtypeStruct((NC * ndstpad, D), jnp.float32),
        mesh=_MESH,
        scratch_types=[
            pltpu.VMEM_SHARED((ndstpad, D), jnp.float32),
            pltpu.VMEM((ZR, D), jnp.float32),
            pltpu.VMEM((K,), jnp.int32),
            pltpu.VMEM((K,), jnp.int32),
            pltpu.VMEM((K, D), jnp.float32),
            pltpu.SemaphoreType.DMA,
        ] + ([pltpu.VMEM((rem,), jnp.int32), pltpu.VMEM((rem,), jnp.int32),
              pltpu.VMEM((rem, D), jnp.float32)] if rem else []),
        compiler_params=_SC_PARAMS,
    )
    def body(gidx_hbm, sidx_hbm, table_hbm, out, acc, zbuf, gi, si, rows,
             sem, *rembufs):
        c = lax.axis_index("c")
        s = lax.axis_index("s")
        wid = s * NC + c
        _zero_fill(zbuf, ZR, D)
        _zero_region(zbuf, ZR, acc, s * zr, zr)
        plsc.subcore_barrier()

        def chunk(j, _):
            base = wid * e_per + j * K
            pltpu.sync_copy(gidx_hbm.at[pl.ds(base, K)], gi)
            pltpu.sync_copy(sidx_hbm.at[pl.ds(base, K)], si)
            pltpu.async_copy(table_hbm.at[gi], rows, sem).wait()
            pltpu.sync_copy(rows, acc.at[si], add=True)
            return 0

        lax.fori_loop(0, n_full, chunk, 0)
        if rem:
            gr, sr, rrows = rembufs
            base = wid * e_per + n_full * K
            pltpu.sync_copy(gidx_hbm.at[pl.ds(base, rem)], gr)
            pltpu.sync_copy(sidx_hbm.at[pl.ds(base, rem)], sr)
            pltpu.async_copy(table_hbm.at[gr], rrows, sem).wait()
            pltpu.sync_copy(rrows, acc.at[sr], add=True)
        plsc.subcore_barrier()
        pltpu.sync_copy(acc.at[pl.ds(s * zr, zr)],
                        out.at[pl.ds(c * ndstpad + s * zr, zr)])

    return body


@functools.cache
def _batch_gather_kernel(nup, nbp):
    b_per = BATCH // NW

    @functools.partial(
        pl.kernel,
        out_type=tuple(
            [jax.ShapeDtypeStruct((BATCH, D), jnp.float32)] * 6
            + [jax.ShapeDtypeStruct((BATCH, L), jnp.float32)] * 2),
        mesh=_MESH,
        scratch_types=[
            pltpu.VMEM((b_per,), jnp.int32),
            pltpu.VMEM((b_per,), jnp.int32),
            pltpu.VMEM((b_per, D), jnp.float32),
            pltpu.VMEM((b_per, D), jnp.float32),
            pltpu.VMEM((b_per, D), jnp.float32),
            pltpu.VMEM((b_per, D), jnp.float32),
            pltpu.VMEM((b_per, D), jnp.float32),
            pltpu.VMEM((b_per, D), jnp.float32),
            pltpu.VMEM((b_per, L), jnp.float32),
            pltpu.VMEM((b_per, L), jnp.float32),
            pltpu.SemaphoreType.DMA,
        ],
        compiler_params=_SC_PARAMS,
    )
    def body(uia, seg0, seg1, bib, uba, ubb, cnt0, cnt1, uidx, bidx,
             o_uui, o_s0, o_s1, o_bib, o_uub, o_ubb, o_c0, o_c1,
             ui_v, bi_v, b0, b1, b2, b3, b4, b5, b6, b7, sem):
        wid = _wid()
        base = wid * b_per
        pltpu.sync_copy(uidx.at[pl.ds(base, b_per)], ui_v)
        pltpu.sync_copy(bidx.at[pl.ds(base, b_per)], bi_v)
        cps = [
            pltpu.async_copy(uia.at[ui_v], b0, sem),
            pltpu.async_copy(seg0.at[bi_v], b1, sem),
            pltpu.async_copy(seg1.at[bi_v], b2, sem),
            pltpu.async_copy(bib.at[bi_v], b3, sem),
            pltpu.async_copy(uba.at[ui_v], b4, sem),
            pltpu.async_copy(ubb.at[bi_v], b5, sem),
            pltpu.async_copy(cnt0.at[bi_v], b6, sem),
            pltpu.async_copy(cnt1.at[bi_v], b7, sem),
        ]
        for cp in cps:
            cp.wait()
        for buf, out in ((b0, o_uui), (b1, o_s0), (b2, o_s1), (b3, o_bib),
                         (b4, o_uub), (b5, o_ubb), (b6, o_c0), (b7, o_c1)):
            pltpu.sync_copy(buf, out.at[pl.ds(base, b_per)])

    return body


_BLK = 256


def _isa_of(deg0_ref, deg1_ref):
    d = deg0_ref[:, :1] + deg1_ref[:, :1]
    return jnp.where(d > 0.0, lax.rsqrt(d), 0.0)


@functools.cache
def _prescale_kernel(npad):
    def body(d0, d1, x, o):
        o[...] = x[...] * _isa_of(d0, d1)

    grid = npad // _BLK
    row = lambda i: (i, 0)
    return pl.pallas_call(
        body,
        grid=(grid,),
        in_specs=[pl.BlockSpec((_BLK, L), row), pl.BlockSpec((_BLK, L), row),
                  pl.BlockSpec((_BLK, D), row)],
        out_specs=pl.BlockSpec((_BLK, D), row),
        out_shape=jax.ShapeDtypeStruct((npad, D), jnp.float32),
    )


@functools.cache
def _post_kernel(npad, final):
    def body(p0, p1, d0, d1, accin, accout, *tilde):
        isa = _isa_of(d0, d1)
        x = isa * (p0[...] + p1[...])
        if final:
            accout[...] = (accin[...] + x) * jnp.float32(1.0 / 3.0)
        else:
            accout[...] = accin[...] + x
            tilde[0][...] = isa * x

    grid = npad // _BLK
    row = lambda i: (i, 0)
    n_out = 1 if final else 2
    return pl.pallas_call(
        body,
        grid=(grid,),
        in_specs=[pl.BlockSpec((_BLK, D), row), pl.BlockSpec((_BLK, D), row),
                  pl.BlockSpec((_BLK, L), row), pl.BlockSpec((_BLK, L), row),
                  pl.BlockSpec((_BLK, D), row)],
        out_specs=[pl.BlockSpec((_BLK, D), row)] * n_out,
        out_shape=[jax.ShapeDtypeStruct((npad, D), jnp.float32)] * n_out,
    )


@functools.cache
def _score_kernel():
    blk = 512
    lam = 1.0 / (1.0 + math.exp(-0.5))

    def body(uui, s0, s1, bib, uub, ubb, c0, c1, o):
        cnt = c0[:, :1] + c1[:, :1]
        b_items = (s0[...] + s1[...]) / (cnt + 1e-8)
        sc = jnp.sum(uui[...] * (b_items + jnp.float32(lam) * bib[...]),
                     axis=-1, keepdims=True)
        sc = sc + jnp.sum(uub[...] * ubb[...], axis=-1, keepdims=True)
        o[...] = sc

    row = lambda i: (i, 0)
    return pl.pallas_call(
        body,
        grid=(BATCH // blk,),
        in_specs=[pl.BlockSpec((blk, D), row)] * 2
        + [pl.BlockSpec((blk, D), row)] * 4
        + [pl.BlockSpec((blk, L), row)] * 2,
        out_specs=pl.BlockSpec((blk, 1), row),
        out_shape=jax.ShapeDtypeStruct((BATCH, 1), jnp.float32),
    )


def _pad_rows(x, npad):
    n = x.shape[0]
    return jnp.pad(x, ((0, npad - n), (0, 0)))


def _lightgcn_sc(a0p, b0p, src, dst, napad, nbpad, E, need_b):
    dega_f, degb_f = _deg_kernel(E, napad, nbpad)(src, dst)
    dega = (dega_f[:napad], dega_f[napad:])
    degb = (degb_f[:nbpad], degb_f[nbpad:])
    at0 = _prescale_kernel(napad)(dega[0], dega[1], a0p)
    bt0 = _prescale_kernel(nbpad)(degb[0], degb[1], b0p)
    pa = _edge_kernel(E, nbpad, napad)(dst, src, bt0)
    pb = _edge_kernel(E, napad, nbpad)(src, dst, at0)
    accA1, at1 = _post_kernel(napad, False)(pa[:napad], pa[napad:],
                                            dega[0], dega[1], a0p)
    accB1, bt1 = _post_kernel(nbpad, False)(pb[:nbpad], pb[nbpad:],
                                            degb[0], degb[1], b0p)
    pa2 = _edge_kernel(E, nbpad, napad)(dst, src, bt1)
    accA = _post_kernel(napad, True)(pa2[:napad], pa2[napad:],
                                     dega[0], dega[1], accA1)[0]
    if need_b:
        pb2 = _edge_kernel(E, napad, nbpad)(src, dst, at1)
        accB = _post_kernel(nbpad, True)(pb2[:nbpad], pb2[nbpad:],
                                         degb[0], degb[1], accB1)[0]
    else:
        accB = None
    return accA, accB, dega


def kernel(users_feature, items_feature, bundles_feature, ui_src, ui_dst,
           bi_src, bi_dst, ub_src, ub_dst, users_idx, bundles_idx):
    uf = _pad_rows(users_feature, N_UP)
    itf = _pad_rows(items_feature, N_IP)
    bf = _pad_rows(bundles_feature, N_BP)

    E_UI = ui_src.shape[0]
    E_BI = bi_src.shape[0]
    E_UB = ub_src.shape[0]

    ui_u, ui_i, _ = _lightgcn_sc(uf, itf, ui_src, ui_dst, N_UP, N_IP,
                                 E_UI, True)
    bi_b, _, bideg = _lightgcn_sc(bf, itf, bi_src, bi_dst, N_BP, N_IP,
                                  E_BI, False)
    ub_u, ub_b, _ = _lightgcn_sc(uf, bf, ub_src, ub_dst, N_UP, N_BP,
                                 E_UB, True)

    seg = _edge_kernel(E_BI, N_IP, N_BP)(bi_dst, bi_src, ui_i)

    outs = _batch_gather_kernel(N_UP, N_BP)(
        ui_u, seg[:N_BP], seg[N_BP:], bi_b, ub_u, ub_b,
        bideg[0], bideg[1], users_idx, bundles_idx)
    score = _score_kernel()(*outs)
    return score[:, 0]


# speedup vs baseline: 25.2954x; 2.1228x over previous
import functools
import math

import jax
import jax.numpy as jnp
from jax import lax
from jax.experimental import pallas as pl
from jax.experimental.pallas import tpu as pltpu
from jax.experimental.pallas import tpu_sc as plsc

NC = 2
NS = 16
NW = NC * NS
L = 16
D = 64
K = 128

N_U = 20000
N_I = 20000
N_B = 10000
BATCH = 4096


def _pad(n):
    m = 8 * NW
    return ((n + m - 1) // m) * m


N_UP = _pad(N_U)
N_IP = _pad(N_I)
N_BP = _pad(N_B)

_MESH = plsc.VectorSubcoreMesh(core_axis_name="c", subcore_axis_name="s")
_SC_PARAMS = pltpu.CompilerParams(use_tc_tiling_on_sc=False)


def _wid():
    return lax.axis_index("s") * NC + lax.axis_index("c")


def _zero_fill(zbuf, rows, cols):
    zv = jnp.zeros((L,), jnp.float32)
    for i in range(rows):
        for j in range(cols // L):
            zbuf[i, pl.ds(j * L, L)] = zv


def _zero_region(zbuf, zrows, acc, base, total):
    full, rem = divmod(total, zrows)
    for t in range(full):
        pltpu.sync_copy(zbuf, acc.at[pl.ds(base + t * zrows, zrows)])
    if rem:
        pltpu.sync_copy(zbuf.at[pl.ds(0, rem)],
                        acc.at[pl.ds(base + full * zrows, rem)])


NBUF = 3


@functools.cache
def _deg_kernel(E, napad, nbpad):
    e_per = E // NW
    n_full, rem = divmod(e_per, K)
    n3 = n_full - (n_full % NBUF)
    tail = n_full - n3
    za = napad // NS
    zb = nbpad // NS
    ZR = 128

    @functools.partial(
        pl.kernel,
        out_type=(jax.ShapeDtypeStruct((NC * napad, L), jnp.float32),
                  jax.ShapeDtypeStruct((NC * nbpad, L), jnp.float32)),
        mesh=_MESH,
        scratch_types=[
            pltpu.VMEM_SHARED((napad, L), jnp.float32),
            pltpu.VMEM_SHARED((nbpad, L), jnp.float32),
            pltpu.VMEM((ZR, L), jnp.float32),
            pltpu.VMEM((K, L), jnp.float32),
        ] + [pltpu.VMEM((K,), jnp.int32) for _ in range(2 * NBUF)]
        + [pltpu.SemaphoreType.DMA for _ in range(4 * NBUF)]
        + ([pltpu.VMEM((rem,), jnp.int32), pltpu.VMEM((rem,), jnp.int32)]
           if rem else []),
        compiler_params=_SC_PARAMS,
    )
    def body(src_hbm, dst_hbm, outa, outb, acc_a, acc_b, zbuf, ones_v,
             *bufs):
        SI = bufs[0:NBUF]
        DI = bufs[NBUF:2 * NBUF]
        sem_is = bufs[2 * NBUF:3 * NBUF]
        sem_id = bufs[3 * NBUF:4 * NBUF]
        sem_sa = bufs[4 * NBUF:5 * NBUF]
        sem_sb = bufs[5 * NBUF:6 * NBUF]
        rembufs = bufs[6 * NBUF:]
        c = lax.axis_index("c")
        s = lax.axis_index("s")
        wid = s * NC + c
        _zero_fill(zbuf, ZR, L)
        ov = jnp.full((L,), 1.0, jnp.float32)
        for i in range(K):
            ones_v[i, :] = ov
        _zero_region(zbuf, ZR, acc_a, s * za, za)
        _zero_region(zbuf, ZR, acc_b, s * zb, zb)
        plsc.subcore_barrier()

        def issue_idx(chunk, b):
            base = wid * e_per + chunk * K
            pltpu.async_copy(src_hbm.at[pl.ds(base, K)], SI[b], sem_is[b])
            pltpu.async_copy(dst_hbm.at[pl.ds(base, K)], DI[b], sem_id[b])

        def wait_idx(b):
            pltpu.make_async_copy(src_hbm.at[pl.ds(0, K)], SI[b],
                                  sem_is[b]).wait()
            pltpu.make_async_copy(dst_hbm.at[pl.ds(0, K)], DI[b],
                                  sem_id[b]).wait()

        def issue_scatter(b):
            pltpu.async_copy(ones_v, acc_a.at[SI[b]], sem_sa[b], add=True)
            pltpu.async_copy(ones_v, acc_b.at[DI[b]], sem_sb[b], add=True)

        def wait_scatter(b):
            pltpu.make_async_copy(outa.at[pl.ds(0, K)], ones_v,
                                  sem_sa[b]).wait()
            pltpu.make_async_copy(outa.at[pl.ds(0, K)], ones_v,
                                  sem_sb[b]).wait()

        if n3:
            issue_idx(0, 0)

            def outer(t, _):
                for kk in range(NBUF):
                    chunk = NBUF * t + kk
                    b = kk
                    bn = (kk + 1) % NBUF
                    wait_idx(b)
                    issue_scatter(b)

                    @pl.when(chunk >= 2)
                    def _():
                        wait_scatter(bn)

                    @pl.when(chunk + 1 < n3)
                    def _():
                        issue_idx(chunk + 1, bn)
                return 0

            lax.fori_loop(0, n3 // NBUF, outer, 0)
            wait_scatter((n3 - 2) % NBUF)
            wait_scatter((n3 - 1) % NBUF)
        for j in range(n3, n3 + tail):
            b = 0
            issue_idx(j, b)
            wait_idx(b)
            issue_scatter(b)
            wait_scatter(b)
        if rem:
            sr, dr = rembufs
            base = wid * e_per + n_full * K
            pltpu.sync_copy(src_hbm.at[pl.ds(base, rem)], sr)
            pltpu.sync_copy(dst_hbm.at[pl.ds(base, rem)], dr)
            pltpu.sync_copy(ones_v.at[pl.ds(0, rem)], acc_a.at[sr], add=True)
            pltpu.sync_copy(ones_v.at[pl.ds(0, rem)], acc_b.at[dr], add=True)
        plsc.subcore_barrier()
        pltpu.sync_copy(acc_a.at[pl.ds(s * za, za)],
                        outa.at[pl.ds(c * napad + s * za, za)])
        pltpu.sync_copy(acc_b.at[pl.ds(s * zb, zb)],
                        outb.at[pl.ds(c * nbpad + s * zb, zb)])

    return body


@functools.cache
def _edge_kernel(E, tabrows, ndstpad):
    e_per = E // NW
    n_full, rem = divmod(e_per, K)
    n3 = n_full - (n_full % NBUF)
    tail = n_full - n3
    zr = ndstpad // NS
    ZR = 128

    @functools.partial(
        pl.kernel,
        out_type=jax.ShapeDtypeStruct((NC * ndstpad, D), jnp.float32),
        mesh=_MESH,
        scratch_types=[
            pltpu.VMEM_SHARED((ndstpad, D), jnp.float32),
            pltpu.VMEM((ZR, D), jnp.float32),
        ] + [pltpu.VMEM((K,), jnp.int32) for _ in range(2 * NBUF)]
        + [pltpu.VMEM((K, D), jnp.float32) for _ in range(NBUF)]
        + [pltpu.SemaphoreType.DMA for _ in range(4 * NBUF)]
        + ([pltpu.VMEM((rem,), jnp.int32), pltpu.VMEM((rem,), jnp.int32),
            pltpu.VMEM((rem, D), jnp.float32), pltpu.SemaphoreType.DMA]
           if rem else []),
        compiler_params=_SC_PARAMS,
    )
    def body(gidx_hbm, sidx_hbm, table_hbm, out, acc, zbuf, *bufs):
        GI = bufs[0:NBUF]
        SI = bufs[NBUF:2 * NBUF]
        RW = bufs[2 * NBUF:3 * NBUF]
        sem_ig = bufs[3 * NBUF:4 * NBUF]
        sem_is = bufs[4 * NBUF:5 * NBUF]
        sem_g = bufs[5 * NBUF:6 * NBUF]
        sem_s = bufs[6 * NBUF:7 * NBUF]
        rembufs = bufs[7 * NBUF:]
        c = lax.axis_index("c")
        s = lax.axis_index("s")
        wid = s * NC + c
        _zero_fill(zbuf, ZR, D)
        _zero_region(zbuf, ZR, acc, s * zr, zr)
        plsc.subcore_barrier()

        def issue_idx(chunk, b):
            base = wid * e_per + chunk * K
            pltpu.async_copy(gidx_hbm.at[pl.ds(base, K)], GI[b], sem_ig[b])
            pltpu.async_copy(sidx_hbm.at[pl.ds(base, K)], SI[b], sem_is[b])

        def wait_idx(b):
            pltpu.make_async_copy(gidx_hbm.at[pl.ds(0, K)], GI[b],
                                  sem_ig[b]).wait()
            pltpu.make_async_copy(sidx_hbm.at[pl.ds(0, K)], SI[b],
                                  sem_is[b]).wait()

        def issue_gather(b):
            pltpu.async_copy(table_hbm.at[GI[b]], RW[b], sem_g[b])

        def wait_gather(b):
            pltpu.make_async_copy(table_hbm.at[pl.ds(0, K)], RW[b],
                                  sem_g[b]).wait()

        def issue_scatter(b):
            pltpu.async_copy(RW[b], acc.at[SI[b]], sem_s[b], add=True)

        def wait_scatter(b):
            pltpu.make_async_copy(table_hbm.at[pl.ds(0, K)], RW[b],
                                  sem_s[b]).wait()

        if n3:
            issue_idx(0, 0)

            def outer(t, _):
                for kk in range(NBUF):
                    chunk = NBUF * t + kk
                    b = kk
                    bn = (kk + 1) % NBUF
                    bp = (kk + 2) % NBUF
                    wait_idx(b)
                    issue_gather(b)

                    @pl.when(chunk >= 1)
                    def _():
                        wait_gather(bp)
                        issue_scatter(bp)

                    @pl.when(chunk >= 2)
                    def _():
                        wait_scatter(bn)

                    @pl.when(chunk + 1 < n3)
                    def _():
                        issue_idx(chunk + 1, bn)
                return 0

            lax.fori_loop(0, n3 // NBUF, outer, 0)
            bl = (n3 - 1) % NBUF
            wait_gather(bl)
            issue_scatter(bl)
            wait_scatter((n3 - 2) % NBUF)
            wait_scatter(bl)
        for j in range(n3, n3 + tail):
            issue_idx(j, 0)
            wait_idx(0)
            issue_gather(0)
            wait_gather(0)
            issue_scatter(0)
            wait_scatter(0)
        if rem:
            gr, sr, rrows, rsem = rembufs
            base = wid * e_per + n_full * K
            pltpu.sync_copy(gidx_hbm.at[pl.ds(base, rem)], gr)
            pltpu.sync_copy(sidx_hbm.at[pl.ds(base, rem)], sr)
            pltpu.async_copy(table_hbm.at[gr], rrows, rsem).wait()
            pltpu.sync_copy(rrows, acc.at[sr], add=True)
        plsc.subcore_barrier()
        pltpu.sync_copy(acc.at[pl.ds(s * zr, zr)],
                        out.at[pl.ds(c * ndstpad + s * zr, zr)])

    return body


@functools.cache
def _batch_gather_kernel(nup, nbp):
    b_per = BATCH // NW

    @functools.partial(
        pl.kernel,
        out_type=tuple(
            [jax.ShapeDtypeStruct((BATCH, D), jnp.float32)] * 6
            + [jax.ShapeDtypeStruct((BATCH, L), jnp.float32)] * 2),
        mesh=_MESH,
        scratch_types=[
            pltpu.VMEM((b_per,), jnp.int32),
            pltpu.VMEM((b_per,), jnp.int32),
            pltpu.VMEM((b_per, D), jnp.float32),
            pltpu.VMEM((b_per, D), jnp.float32),
            pltpu.VMEM((b_per, D), jnp.float32),
            pltpu.VMEM((b_per, D), jnp.float32),
            pltpu.VMEM((b_per, D), jnp.float32),
            pltpu.VMEM((b_per, D), jnp.float32),
            pltpu.VMEM((b_per, L), jnp.float32),
            pltpu.VMEM((b_per, L), jnp.float32),
            pltpu.SemaphoreType.DMA,
        ],
        compiler_params=_SC_PARAMS,
    )
    def body(uia, seg0, seg1, bib, uba, ubb, cnt0, cnt1, uidx, bidx,
             o_uui, o_s0, o_s1, o_bib, o_uub, o_ubb, o_c0, o_c1,
             ui_v, bi_v, b0, b1, b2, b3, b4, b5, b6, b7, sem):
        wid = _wid()
        base = wid * b_per
        pltpu.sync_copy(uidx.at[pl.ds(base, b_per)], ui_v)
        pltpu.sync_copy(bidx.at[pl.ds(base, b_per)], bi_v)
        cps = [
            pltpu.async_copy(uia.at[ui_v], b0, sem),
            pltpu.async_copy(seg0.at[bi_v], b1, sem),
            pltpu.async_copy(seg1.at[bi_v], b2, sem),
            pltpu.async_copy(bib.at[bi_v], b3, sem),
            pltpu.async_copy(uba.at[ui_v], b4, sem),
            pltpu.async_copy(ubb.at[bi_v], b5, sem),
            pltpu.async_copy(cnt0.at[bi_v], b6, sem),
            pltpu.async_copy(cnt1.at[bi_v], b7, sem),
        ]
        for cp in cps:
            cp.wait()
        for buf, out in ((b0, o_uui), (b1, o_s0), (b2, o_s1), (b3, o_bib),
                         (b4, o_uub), (b5, o_ubb), (b6, o_c0), (b7, o_c1)):
            pltpu.sync_copy(buf, out.at[pl.ds(base, b_per)])

    return body


_BLK = 256


def _isa_of(deg0_ref, deg1_ref):
    d = deg0_ref[:, :1] + deg1_ref[:, :1]
    return jnp.where(d > 0.0, lax.rsqrt(d), 0.0)


@functools.cache
def _prescale_kernel(npad):
    def body(d0, d1, x, o):
        o[...] = x[...] * _isa_of(d0, d1)

    grid = npad // _BLK
    row = lambda i: (i, 0)
    return pl.pallas_call(
        body,
        grid=(grid,),
        in_specs=[pl.BlockSpec((_BLK, L), row), pl.BlockSpec((_BLK, L), row),
                  pl.BlockSpec((_BLK, D), row)],
        out_specs=pl.BlockSpec((_BLK, D), row),
        out_shape=jax.ShapeDtypeStruct((npad, D), jnp.float32),
    )


@functools.cache
def _post_kernel(npad, final):
    def body(p0, p1, d0, d1, accin, accout, *tilde):
        isa = _isa_of(d0, d1)
        x = isa * (p0[...] + p1[...])
        if final:
            accout[...] = (accin[...] + x) * jnp.float32(1.0 / 3.0)
        else:
            accout[...] = accin[...] + x
            tilde[0][...] = isa * x

    grid = npad // _BLK
    row = lambda i: (i, 0)
    n_out = 1 if final else 2
    return pl.pallas_call(
        body,
        grid=(grid,),
        in_specs=[pl.BlockSpec((_BLK, D), row), pl.BlockSpec((_BLK, D), row),
                  pl.BlockSpec((_BLK, L), row), pl.BlockSpec((_BLK, L), row),
                  pl.BlockSpec((_BLK, D), row)],
        out_specs=[pl.BlockSpec((_BLK, D), row)] * n_out,
        out_shape=[jax.ShapeDtypeStruct((npad, D), jnp.float32)] * n_out,
    )


@functools.cache
def _score_kernel():
    blk = 512
    lam = 1.0 / (1.0 + math.exp(-0.5))

    def body(uui, s0, s1, bib, uub, ubb, c0, c1, o):
        cnt = c0[:, :1] + c1[:, :1]
        b_items = (s0[...] + s1[...]) / (cnt + 1e-8)
        sc = jnp.sum(uui[...] * (b_items + jnp.float32(lam) * bib[...]),
                     axis=-1, keepdims=True)
        sc = sc + jnp.sum(uub[...] * ubb[...], axis=-1, keepdims=True)
        o[...] = sc

    row = lambda i: (i, 0)
    return pl.pallas_call(
        body,
        grid=(BATCH // blk,),
        in_specs=[pl.BlockSpec((blk, D), row)] * 2
        + [pl.BlockSpec((blk, D), row)] * 4
        + [pl.BlockSpec((blk, L), row)] * 2,
        out_specs=pl.BlockSpec((blk, 1), row),
        out_shape=jax.ShapeDtypeStruct((BATCH, 1), jnp.float32),
    )


def _pad_rows(x, npad):
    n = x.shape[0]
    return jnp.pad(x, ((0, npad - n), (0, 0)))


def _lightgcn_sc(a0p, b0p, src, dst, napad, nbpad, E, need_b):
    dega_f, degb_f = _deg_kernel(E, napad, nbpad)(src, dst)
    dega = (dega_f[:napad], dega_f[napad:])
    degb = (degb_f[:nbpad], degb_f[nbpad:])
    at0 = _prescale_kernel(napad)(dega[0], dega[1], a0p)
    bt0 = _prescale_kernel(nbpad)(degb[0], degb[1], b0p)
    pa = _edge_kernel(E, nbpad, napad)(dst, src, bt0)
    pb = _edge_kernel(E, napad, nbpad)(src, dst, at0)
    accA1, at1 = _post_kernel(napad, False)(pa[:napad], pa[napad:],
                                            dega[0], dega[1], a0p)
    accB1, bt1 = _post_kernel(nbpad, False)(pb[:nbpad], pb[nbpad:],
                                            degb[0], degb[1], b0p)
    pa2 = _edge_kernel(E, nbpad, napad)(dst, src, bt1)
    accA = _post_kernel(napad, True)(pa2[:napad], pa2[napad:],
                                     dega[0], dega[1], accA1)[0]
    if need_b:
        pb2 = _edge_kernel(E, napad, nbpad)(src, dst, at1)
        accB = _post_kernel(nbpad, True)(pb2[:nbpad], pb2[nbpad:],
                                         degb[0], degb[1], accB1)[0]
    else:
        accB = None
    return accA, accB, dega


def kernel(users_feature, items_feature, bundles_feature, ui_src, ui_dst,
           bi_src, bi_dst, ub_src, ub_dst, users_idx, bundles_idx):
    uf = _pad_rows(users_feature, N_UP)
    itf = _pad_rows(items_feature, N_IP)
    bf = _pad_rows(bundles_feature, N_BP)

    E_UI = ui_src.shape[0]
    E_BI = bi_src.shape[0]
    E_UB = ub_src.shape[0]

    ui_u, ui_i, _ = _lightgcn_sc(uf, itf, ui_src, ui_dst, N_UP, N_IP,
                                 E_UI, True)
    bi_b, _, bideg = _lightgcn_sc(bf, itf, bi_src, bi_dst, N_BP, N_IP,
                                  E_BI, False)
    ub_u, ub_b, _ = _lightgcn_sc(uf, bf, ub_src, ub_dst, N_UP, N_BP,
                                 E_UB, True)

    seg = _edge_kernel(E_BI, N_IP, N_BP)(bi_dst, bi_src, ui_i)

    outs = _batch_gather_kernel(N_UP, N_BP)(
        ui_u, seg[:N_BP], seg[N_BP:], bi_b, ub_u, ub_b,
        bideg[0], bideg[1], users_idx, bundles_idx)
    score = _score_kernel()(*outs)
    return score[:, 0]


# speedup vs baseline: 26.0353x; 1.0292x over previous
import functools
import math

import jax
import jax.numpy as jnp
from jax import lax
from jax.experimental import pallas as pl
from jax.experimental.pallas import tpu as pltpu
from jax.experimental.pallas import tpu_sc as plsc

NC = 2
NS = 16
NW = NC * NS
L = 16
D = 64
K = 128

N_U = 20000
N_I = 20000
N_B = 10000
BATCH = 4096


def _pad(n):
    m = 8 * NW
    return ((n + m - 1) // m) * m


N_UP = _pad(N_U)
N_IP = _pad(N_I)
N_BP = _pad(N_B)

_MESH = plsc.VectorSubcoreMesh(core_axis_name="c", subcore_axis_name="s")
_SC_PARAMS = pltpu.CompilerParams(use_tc_tiling_on_sc=False)


def _wid():
    return lax.axis_index("s") * NC + lax.axis_index("c")


def _zero_fill(zbuf, rows, cols):
    zv = jnp.zeros((L,), jnp.float32)
    for i in range(rows):
        for j in range(cols // L):
            zbuf[i, pl.ds(j * L, L)] = zv


def _zero_region(zbuf, zrows, acc, base, total):
    full, rem = divmod(total, zrows)
    for t in range(full):
        pltpu.sync_copy(zbuf, acc.at[pl.ds(base + t * zrows, zrows)])
    if rem:
        pltpu.sync_copy(zbuf.at[pl.ds(0, rem)],
                        acc.at[pl.ds(base + full * zrows, rem)])


NBUF = 3


@functools.cache
def _deg_kernel(E, napad, nbpad):
    e_per = E // NW
    n_full, rem = divmod(e_per, K)
    n3 = n_full - (n_full % NBUF)
    tail = n_full - n3
    za = napad // NS
    zb = nbpad // NS
    ZR = 128

    @functools.partial(
        pl.kernel,
        out_type=(jax.ShapeDtypeStruct((NC * napad, L), jnp.float32),
                  jax.ShapeDtypeStruct((NC * nbpad, L), jnp.float32)),
        mesh=_MESH,
        scratch_types=[
            pltpu.VMEM_SHARED((napad, L), jnp.float32),
            pltpu.VMEM_SHARED((nbpad, L), jnp.float32),
            pltpu.VMEM((ZR, L), jnp.float32),
            pltpu.VMEM((K, L), jnp.float32),
        ] + [pltpu.VMEM((K,), jnp.int32) for _ in range(2 * NBUF)]
        + [pltpu.SemaphoreType.DMA for _ in range(4 * NBUF)]
        + ([pltpu.VMEM((rem,), jnp.int32), pltpu.VMEM((rem,), jnp.int32)]
           if rem else []),
        compiler_params=_SC_PARAMS,
    )
    def body(src_hbm, dst_hbm, outa, outb, acc_a, acc_b, zbuf, ones_v,
             *bufs):
        SI = bufs[0:NBUF]
        DI = bufs[NBUF:2 * NBUF]
        sem_is = bufs[2 * NBUF:3 * NBUF]
        sem_id = bufs[3 * NBUF:4 * NBUF]
        sem_sa = bufs[4 * NBUF:5 * NBUF]
        sem_sb = bufs[5 * NBUF:6 * NBUF]
        rembufs = bufs[6 * NBUF:]
        c = lax.axis_index("c")
        s = lax.axis_index("s")
        wid = s * NC + c
        _zero_fill(zbuf, ZR, L)
        ov = jnp.full((L,), 1.0, jnp.float32)
        for i in range(K):
            ones_v[i, :] = ov
        _zero_region(zbuf, ZR, acc_a, s * za, za)
        _zero_region(zbuf, ZR, acc_b, s * zb, zb)
        plsc.subcore_barrier()

        def issue_idx(chunk, b):
            base = wid * e_per + chunk * K
            pltpu.async_copy(src_hbm.at[pl.ds(base, K)], SI[b], sem_is[b])
            pltpu.async_copy(dst_hbm.at[pl.ds(base, K)], DI[b], sem_id[b])

        def wait_idx(b):
            pltpu.make_async_copy(src_hbm.at[pl.ds(0, K)], SI[b],
                                  sem_is[b]).wait()
            pltpu.make_async_copy(dst_hbm.at[pl.ds(0, K)], DI[b],
                                  sem_id[b]).wait()

        def issue_scatter(b):
            pltpu.async_copy(ones_v, acc_a.at[SI[b]], sem_sa[b], add=True)
            pltpu.async_copy(ones_v, acc_b.at[DI[b]], sem_sb[b], add=True)

        def wait_scatter(b):
            pltpu.make_async_copy(outa.at[pl.ds(0, K)], ones_v,
                                  sem_sa[b]).wait()
            pltpu.make_async_copy(outa.at[pl.ds(0, K)], ones_v,
                                  sem_sb[b]).wait()

        if n3:
            issue_idx(0, 0)

            def outer(t, _):
                for kk in range(NBUF):
                    chunk = NBUF * t + kk
                    b = kk
                    bn = (kk + 1) % NBUF
                    wait_idx(b)
                    issue_scatter(b)

                    @pl.when(chunk >= 2)
                    def _():
                        wait_scatter(bn)

                    @pl.when(chunk + 1 < n3)
                    def _():
                        issue_idx(chunk + 1, bn)
                return 0

            lax.fori_loop(0, n3 // NBUF, outer, 0)
            wait_scatter((n3 - 2) % NBUF)
            wait_scatter((n3 - 1) % NBUF)
        for j in range(n3, n3 + tail):
            b = 0
            issue_idx(j, b)
            wait_idx(b)
            issue_scatter(b)
            wait_scatter(b)
        if rem:
            sr, dr = rembufs
            base = wid * e_per + n_full * K
            pltpu.sync_copy(src_hbm.at[pl.ds(base, rem)], sr)
            pltpu.sync_copy(dst_hbm.at[pl.ds(base, rem)], dr)
            pltpu.sync_copy(ones_v.at[pl.ds(0, rem)], acc_a.at[sr], add=True)
            pltpu.sync_copy(ones_v.at[pl.ds(0, rem)], acc_b.at[dr], add=True)
        plsc.subcore_barrier()
        pltpu.sync_copy(acc_a.at[pl.ds(s * za, za)],
                        outa.at[pl.ds(c * napad + s * za, za)])
        pltpu.sync_copy(acc_b.at[pl.ds(s * zb, zb)],
                        outb.at[pl.ds(c * nbpad + s * zb, zb)])

    return body


@functools.cache
def _edge_kernel(E, tabrows, ndstpad):
    NB = 4 if ndstpad >= 16000 else 6
    WLAG = 3 if NB == 4 else 4
    e_per = E // NW
    n_full, rem = divmod(e_per, K)
    n3 = n_full - (n_full % NB)
    tail = n_full - n3
    zr = ndstpad // NS
    ZR = 64

    @functools.partial(
        pl.kernel,
        out_type=jax.ShapeDtypeStruct((NC * ndstpad, D), jnp.float32),
        mesh=_MESH,
        scratch_types=[
            pltpu.VMEM_SHARED((ndstpad, D), jnp.float32),
            pltpu.VMEM((ZR, D), jnp.float32),
        ] + [pltpu.VMEM((K,), jnp.int32) for _ in range(2 * NB)]
        + [pltpu.VMEM((K, D), jnp.float32) for _ in range(NB)]
        + [pltpu.SemaphoreType.DMA for _ in range(4 * NB)]
        + ([pltpu.VMEM((rem,), jnp.int32), pltpu.VMEM((rem,), jnp.int32),
            pltpu.VMEM((rem, D), jnp.float32), pltpu.SemaphoreType.DMA]
           if rem else []),
        compiler_params=_SC_PARAMS,
    )
    def body(gidx_hbm, sidx_hbm, table_hbm, out, acc, zbuf, *bufs):
        GI = bufs[0:NB]
        SI = bufs[NB:2 * NB]
        RW = bufs[2 * NB:3 * NB]
        sem_ig = bufs[3 * NB:4 * NB]
        sem_is = bufs[4 * NB:5 * NB]
        sem_g = bufs[5 * NB:6 * NB]
        sem_s = bufs[6 * NB:7 * NB]
        rembufs = bufs[7 * NB:]
        c = lax.axis_index("c")
        s = lax.axis_index("s")
        wid = s * NC + c
        _zero_fill(zbuf, ZR, D)
        _zero_region(zbuf, ZR, acc, s * zr, zr)
        plsc.subcore_barrier()

        def issue_idx(chunk, b):
            base = wid * e_per + chunk * K
            pltpu.async_copy(gidx_hbm.at[pl.ds(base, K)], GI[b], sem_ig[b])
            pltpu.async_copy(sidx_hbm.at[pl.ds(base, K)], SI[b], sem_is[b])

        def wait_idx(b):
            pltpu.make_async_copy(gidx_hbm.at[pl.ds(0, K)], GI[b],
                                  sem_ig[b]).wait()
            pltpu.make_async_copy(sidx_hbm.at[pl.ds(0, K)], SI[b],
                                  sem_is[b]).wait()

        def issue_gather(b):
            pltpu.async_copy(table_hbm.at[GI[b]], RW[b], sem_g[b])

        def wait_gather(b):
            pltpu.make_async_copy(table_hbm.at[pl.ds(0, K)], RW[b],
                                  sem_g[b]).wait()

        def issue_scatter(b):
            pltpu.async_copy(RW[b], acc.at[SI[b]], sem_s[b], add=True)

        def wait_scatter(b):
            pltpu.make_async_copy(table_hbm.at[pl.ds(0, K)], RW[b],
                                  sem_s[b]).wait()

        if n3:
            issue_idx(0, 0)

            def outer(t, _):
                for kk in range(NB):
                    chunk = NB * t + kk
                    b = kk
                    bn = (kk + 1) % NB
                    b2 = (kk - 2) % NB
                    bw = (kk - WLAG) % NB
                    wait_idx(b)
                    issue_gather(b)

                    @pl.when(chunk >= 2)
                    def _():
                        wait_gather(b2)
                        issue_scatter(b2)

                    @pl.when(chunk >= WLAG)
                    def _():
                        wait_scatter(bw)

                    @pl.when(chunk + 1 < n3)
                    def _():
                        issue_idx(chunk + 1, bn)
                return 0

            lax.fori_loop(0, n3 // NB, outer, 0)
            for chunk in (n3 - 2, n3 - 1):
                wait_gather(chunk % NB)
                issue_scatter(chunk % NB)
            for chunk in range(n3 - WLAG, n3):
                wait_scatter(chunk % NB)
        for j in range(n3, n3 + tail):
            issue_idx(j, 0)
            wait_idx(0)
            issue_gather(0)
            wait_gather(0)
            issue_scatter(0)
            wait_scatter(0)
        if rem:
            gr, sr, rrows, rsem = rembufs
            base = wid * e_per + n_full * K
            pltpu.sync_copy(gidx_hbm.at[pl.ds(base, rem)], gr)
            pltpu.sync_copy(sidx_hbm.at[pl.ds(base, rem)], sr)
            pltpu.async_copy(table_hbm.at[gr], rrows, rsem).wait()
            pltpu.sync_copy(rrows, acc.at[sr], add=True)
        plsc.subcore_barrier()
        pltpu.sync_copy(acc.at[pl.ds(s * zr, zr)],
                        out.at[pl.ds(c * ndstpad + s * zr, zr)])

    return body


@functools.cache
def _batch_gather_kernel(nup, nbp):
    b_per = BATCH // NW

    @functools.partial(
        pl.kernel,
        out_type=tuple(
            [jax.ShapeDtypeStruct((BATCH, D), jnp.float32)] * 6
            + [jax.ShapeDtypeStruct((BATCH, L), jnp.float32)] * 2),
        mesh=_MESH,
        scratch_types=[
            pltpu.VMEM((b_per,), jnp.int32),
            pltpu.VMEM((b_per,), jnp.int32),
            pltpu.VMEM((b_per, D), jnp.float32),
            pltpu.VMEM((b_per, D), jnp.float32),
            pltpu.VMEM((b_per, D), jnp.float32),
            pltpu.VMEM((b_per, D), jnp.float32),
            pltpu.VMEM((b_per, D), jnp.float32),
            pltpu.VMEM((b_per, D), jnp.float32),
            pltpu.VMEM((b_per, L), jnp.float32),
            pltpu.VMEM((b_per, L), jnp.float32),
            pltpu.SemaphoreType.DMA,
        ],
        compiler_params=_SC_PARAMS,
    )
    def body(uia, seg0, seg1, bib, uba, ubb, cnt0, cnt1, uidx, bidx,
             o_uui, o_s0, o_s1, o_bib, o_uub, o_ubb, o_c0, o_c1,
             ui_v, bi_v, b0, b1, b2, b3, b4, b5, b6, b7, sem):
        wid = _wid()
        base = wid * b_per
        pltpu.sync_copy(uidx.at[pl.ds(base, b_per)], ui_v)
        pltpu.sync_copy(bidx.at[pl.ds(base, b_per)], bi_v)
        cps = [
            pltpu.async_copy(uia.at[ui_v], b0, sem),
            pltpu.async_copy(seg0.at[bi_v], b1, sem),
            pltpu.async_copy(seg1.at[bi_v], b2, sem),
            pltpu.async_copy(bib.at[bi_v], b3, sem),
            pltpu.async_copy(uba.at[ui_v], b4, sem),
            pltpu.async_copy(ubb.at[bi_v], b5, sem),
            pltpu.async_copy(cnt0.at[bi_v], b6, sem),
            pltpu.async_copy(cnt1.at[bi_v], b7, sem),
        ]
        for cp in cps:
            cp.wait()
        for buf, out in ((b0, o_uui), (b1, o_s0), (b2, o_s1), (b3, o_bib),
                         (b4, o_uub), (b5, o_ubb), (b6, o_c0), (b7, o_c1)):
            pltpu.sync_copy(buf, out.at[pl.ds(base, b_per)])

    return body


_BLK = 256


def _isa_of(deg0_ref, deg1_ref):
    d = deg0_ref[:, :1] + deg1_ref[:, :1]
    return jnp.where(d > 0.0, lax.rsqrt(d), 0.0)


@functools.cache
def _prescale_kernel(npad):
    def body(d0, d1, x, o):
        o[...] = x[...] * _isa_of(d0, d1)

    grid = npad // _BLK
    row = lambda i: (i, 0)
    return pl.pallas_call(
        body,
        grid=(grid,),
        in_specs=[pl.BlockSpec((_BLK, L), row), pl.BlockSpec((_BLK, L), row),
                  pl.BlockSpec((_BLK, D), row)],
        out_specs=pl.BlockSpec((_BLK, D), row),
        out_shape=jax.ShapeDtypeStruct((npad, D), jnp.float32),
    )


@functools.cache
def _post_kernel(npad, final):
    def body(p0, p1, d0, d1, accin, accout, *tilde):
        isa = _isa_of(d0, d1)
        x = isa * (p0[...] + p1[...])
        if final:
            accout[...] = (accin[...] + x) * jnp.float32(1.0 / 3.0)
        else:
            accout[...] = accin[...] + x
            tilde[0][...] = isa * x

    grid = npad // _BLK
    row = lambda i: (i, 0)
    n_out = 1 if final else 2
    return pl.pallas_call(
        body,
        grid=(grid,),
        in_specs=[pl.BlockSpec((_BLK, D), row), pl.BlockSpec((_BLK, D), row),
                  pl.BlockSpec((_BLK, L), row), pl.BlockSpec((_BLK, L), row),
                  pl.BlockSpec((_BLK, D), row)],
        out_specs=[pl.BlockSpec((_BLK, D), row)] * n_out,
        out_shape=[jax.ShapeDtypeStruct((npad, D), jnp.float32)] * n_out,
    )


@functools.cache
def _score_kernel():
    blk = 512
    lam = 1.0 / (1.0 + math.exp(-0.5))

    def body(uui, s0, s1, bib, uub, ubb, c0, c1, o):
        cnt = c0[:, :1] + c1[:, :1]
        b_items = (s0[...] + s1[...]) / (cnt + 1e-8)
        sc = jnp.sum(uui[...] * (b_items + jnp.float32(lam) * bib[...]),
                     axis=-1, keepdims=True)
        sc = sc + jnp.sum(uub[...] * ubb[...], axis=-1, keepdims=True)
        o[...] = sc

    row = lambda i: (i, 0)
    return pl.pallas_call(
        body,
        grid=(BATCH // blk,),
        in_specs=[pl.BlockSpec((blk, D), row)] * 2
        + [pl.BlockSpec((blk, D), row)] * 4
        + [pl.BlockSpec((blk, L), row)] * 2,
        out_specs=pl.BlockSpec((blk, 1), row),
        out_shape=jax.ShapeDtypeStruct((BATCH, 1), jnp.float32),
    )


def _pad_rows(x, npad):
    n = x.shape[0]
    return jnp.pad(x, ((0, npad - n), (0, 0)))


def _lightgcn_sc(a0p, b0p, src, dst, napad, nbpad, E, need_b):
    dega_f, degb_f = _deg_kernel(E, napad, nbpad)(src, dst)
    dega = (dega_f[:napad], dega_f[napad:])
    degb = (degb_f[:nbpad], degb_f[nbpad:])
    at0 = _prescale_kernel(napad)(dega[0], dega[1], a0p)
    bt0 = _prescale_kernel(nbpad)(degb[0], degb[1], b0p)
    pa = _edge_kernel(E, nbpad, napad)(dst, src, bt0)
    pb = _edge_kernel(E, napad, nbpad)(src, dst, at0)
    accA1, at1 = _post_kernel(napad, False)(pa[:napad], pa[napad:],
                                            dega[0], dega[1], a0p)
    accB1, bt1 = _post_kernel(nbpad, False)(pb[:nbpad], pb[nbpad:],
                                            degb[0], degb[1], b0p)
    pa2 = _edge_kernel(E, nbpad, napad)(dst, src, bt1)
    accA = _post_kernel(napad, True)(pa2[:napad], pa2[napad:],
                                     dega[0], dega[1], accA1)[0]
    if need_b:
        pb2 = _edge_kernel(E, napad, nbpad)(src, dst, at1)
        accB = _post_kernel(nbpad, True)(pb2[:nbpad], pb2[nbpad:],
                                         degb[0], degb[1], accB1)[0]
    else:
        accB = None
    return accA, accB, dega


def kernel(users_feature, items_feature, bundles_feature, ui_src, ui_dst,
           bi_src, bi_dst, ub_src, ub_dst, users_idx, bundles_idx):
    uf = _pad_rows(users_feature, N_UP)
    itf = _pad_rows(items_feature, N_IP)
    bf = _pad_rows(bundles_feature, N_BP)

    E_UI = ui_src.shape[0]
    E_BI = bi_src.shape[0]
    E_UB = ub_src.shape[0]

    ui_u, ui_i, _ = _lightgcn_sc(uf, itf, ui_src, ui_dst, N_UP, N_IP,
                                 E_UI, True)
    bi_b, _, bideg = _lightgcn_sc(bf, itf, bi_src, bi_dst, N_BP, N_IP,
                                  E_BI, False)
    ub_u, ub_b, _ = _lightgcn_sc(uf, bf, ub_src, ub_dst, N_UP, N_BP,
                                 E_UB, True)

    seg = _edge_kernel(E_BI, N_IP, N_BP)(bi_dst, bi_src, ui_i)

    outs = _batch_gather_kernel(N_UP, N_BP)(
        ui_u, seg[:N_BP], seg[N_BP:], bi_b, ub_u, ub_b,
        bideg[0], bideg[1], users_idx, bundles_idx)
    score = _score_kernel()(*outs)
    return score[:, 0]
